# baseline (speedup 1.0000x reference)
import functools
import math

import jax
import jax.numpy as jnp
import numpy as np
from jax import lax
from jax.experimental import pallas as pl
from jax.experimental.pallas import tpu as pltpu
from jax.experimental.pallas import tpu_sc as plsc

VOCAB = 1000000
D_MODEL = 64
MAX_LEN = 512
BATCH = 4096
SEQ = 200
SCALE = math.sqrt(D_MODEL)

NC, NS = 2, 16
NW = NC * NS
FLAT = BATCH * SEQ
PER_W = FLAT // NW
SEQ_PER_W = PER_W // SEQ
LANES = 16
VREGS_PER_ROW = D_MODEL // LANES


def _make_pe():
    pe = np.zeros((MAX_LEN, D_MODEL), dtype=np.float32)
    position = np.arange(0, MAX_LEN, dtype=np.float32)[:, None]
    div_term = np.exp(
        np.arange(0, D_MODEL, 2, dtype=np.float32) * (-math.log(10000.0) / D_MODEL)
    )
    pe[:, 0::2] = np.sin(position * div_term)
    pe[:, 1::2] = np.cos(position * div_term)
    return pe[:SEQ]


_PE_NP = _make_pe()

_MESH = plsc.VectorSubcoreMesh(
    core_axis_name="c", subcore_axis_name="s", num_cores=NC, num_subcores=NS
)


@functools.partial(
    pl.kernel,
    out_type=jax.ShapeDtypeStruct((FLAT, D_MODEL), jnp.float32),
    mesh=_MESH,
    compiler_params=pltpu.CompilerParams(use_tc_tiling_on_sc=False),
    scratch_types=[
        pltpu.VMEM((SEQ,), jnp.int32),
        pltpu.VMEM((SEQ, D_MODEL), jnp.float32),
        pltpu.VMEM((SEQ, D_MODEL), jnp.float32),
        pltpu.SemaphoreType.DMA,
    ],
)
def _embed_kernel(table_hbm, xf_hbm, pe_hbm, out_hbm, idx_v, rows_v, pe_v, sem):
    wid = lax.axis_index("s") * NC + lax.axis_index("c")
    base = wid * PER_W

    pltpu.sync_copy(pe_hbm, pe_v)

    def seq_body(s, _):
        off = base + s * SEQ
        pltpu.sync_copy(xf_hbm.at[pl.ds(off, SEQ)], idx_v)
        pltpu.async_copy(table_hbm.at[idx_v], rows_v, sem).wait()

        @plsc.parallel_loop(0, SEQ, unroll=4)
        def _ew(j):
            for k in range(VREGS_PER_ROW):
                sl = pl.ds(k * LANES, LANES)
                rows_v[j, sl] = rows_v[j, sl] * SCALE + pe_v[j, sl]

        pltpu.sync_copy(rows_v, out_hbm.at[pl.ds(off, SEQ)])
        return 0

    lax.fori_loop(0, SEQ_PER_W, seq_body, 0)


def kernel(x, table):
    xf = x.reshape(FLAT)
    out = _embed_kernel(table, xf, jnp.asarray(_PE_NP))
    return out.reshape(BATCH, SEQ, D_MODEL)


# speedup vs baseline: 1.2048x; 1.2048x over previous
import functools
import math

import jax
import jax.numpy as jnp
import numpy as np
from jax import lax
from jax.experimental import pallas as pl
from jax.experimental.pallas import tpu as pltpu
from jax.experimental.pallas import tpu_sc as plsc

VOCAB = 1000000
D_MODEL = 64
MAX_LEN = 512
BATCH = 4096
SEQ = 200
SCALE = math.sqrt(D_MODEL)

NC, NS = 2, 16
NW = NC * NS
ROWS_PER_W = BATCH // NW
CH = 2
NCHUNK = ROWS_PER_W // CH
CHUNK_TOK = CH * SEQ
LANES = 16
NBUF = 3


def _make_pe():
    pe = np.zeros((MAX_LEN, D_MODEL), dtype=np.float32)
    position = np.arange(0, MAX_LEN, dtype=np.float32)[:, None]
    div_term = np.exp(
        np.arange(0, D_MODEL, 2, dtype=np.float32) * (-math.log(10000.0) / D_MODEL)
    )
    pe[:, 0::2] = np.sin(position * div_term)
    pe[:, 1::2] = np.cos(position * div_term)
    return pe[:SEQ]


_PE_NP = _make_pe()

_MESH = plsc.VectorSubcoreMesh(
    core_axis_name="c", subcore_axis_name="s", num_cores=NC, num_subcores=NS
)


@functools.partial(
    pl.kernel,
    out_type=jax.ShapeDtypeStruct((BATCH, SEQ, D_MODEL), jnp.float32),
    mesh=_MESH,
    compiler_params=pltpu.CompilerParams(use_tc_tiling_on_sc=False),
    scratch_types=[
        pltpu.VMEM((ROWS_PER_W * SEQ,), jnp.int32),
        pltpu.VMEM((CHUNK_TOK, D_MODEL), jnp.float32),
        pltpu.VMEM((CHUNK_TOK, D_MODEL), jnp.float32),
        pltpu.VMEM((CHUNK_TOK, D_MODEL), jnp.float32),
        pltpu.VMEM((SEQ, D_MODEL), jnp.float32),
        pltpu.SemaphoreType.DMA,
        pltpu.SemaphoreType.DMA,
        pltpu.SemaphoreType.DMA,
        pltpu.SemaphoreType.DMA,
        pltpu.SemaphoreType.DMA,
        pltpu.SemaphoreType.DMA,
    ],
)
def _embed_kernel(table_hbm, x_hbm, pe_hbm, out_hbm,
                  idx_all, rows0, rows1, rows2, pe_v,
                  g0, g1, g2, w0, w1, w2):
    wid = lax.axis_index("s") * NC + lax.axis_index("c")
    row0 = wid * ROWS_PER_W
    tok0 = row0 * SEQ

    rows = [rows0, rows1, rows2]
    gsem = [g0, g1, g2]
    wsem = [w0, w1, w2]

    pltpu.sync_copy(pe_hbm, pe_v)
    pltpu.sync_copy(x_hbm.at[pl.ds(tok0, ROWS_PER_W * SEQ)], idx_all)

    def gather_desc(c, k):
        src = table_hbm.at[idx_all.at[pl.ds(c * CHUNK_TOK, CHUNK_TOK)]]
        return pltpu.make_async_copy(src, rows[k], gsem[k])

    def wb_desc(c, k, r):
        b = row0 + c * CH + r
        return pltpu.make_async_copy(
            rows[k].at[pl.ds(r * SEQ, SEQ)], out_hbm.at[b], wsem[k]
        )

    def fire_gather(c, k):
        gather_desc(c, k).start()

    def process(c, k):
        gather_desc(c, k).wait()

        @plsc.parallel_loop(0, SEQ, unroll=2)
        def _ew(j):
            buf = rows[k]
            for q in range(D_MODEL // LANES):
                sl = pl.ds(q * LANES, LANES)
                pe_q = pe_v[j, sl]
                for r in range(CH):
                    buf[r * SEQ + j, sl] = buf[r * SEQ + j, sl] * SCALE + pe_q

        for r in range(CH):
            wb_desc(c, k, r).start()

    def drain_wb(c, k):
        for r in range(CH):
            wb_desc(c, k, r).wait()

    fire_gather(0, 0)
    fire_gather(1, 1)

    process(0, 0)
    fire_gather(2, 2)

    process(1, 1)
    drain_wb(0, 0)
    fire_gather(3, 0)

    process(2, 2)
    drain_wb(1, 1)
    fire_gather(4, 1)

    @pl.loop(1, NCHUNK // NBUF)
    def _pipe(it):
        cbase = it * NBUF
        for k in range(NBUF):
            c = cbase + k
            process(c, k)
            drain_wb(c - 1, (k + NBUF - 1) % NBUF)

            @pl.when(c + 2 < NCHUNK)
            def _():
                fire_gather(c + 2, (k + 2) % NBUF)

    process(NCHUNK - 1, (NCHUNK - 1) % NBUF)
    drain_wb(NCHUNK - 2, (NCHUNK - 2) % NBUF)
    drain_wb(NCHUNK - 1, (NCHUNK - 1) % NBUF)


def kernel(x, table):
    xf = x.reshape(BATCH * SEQ)
    return _embed_kernel(table, xf, jnp.asarray(_PE_NP))


# speedup vs baseline: 1.3620x; 1.1305x over previous
import functools
import math

import jax
import jax.numpy as jnp
import numpy as np
from jax import lax
from jax.experimental import pallas as pl
from jax.experimental.pallas import tpu as pltpu
from jax.experimental.pallas import tpu_sc as plsc

VOCAB = 1000000
D_MODEL = 64
MAX_LEN = 512
BATCH = 4096
SEQ = 200
SCALE = math.sqrt(D_MODEL)

NC, NS = 2, 16
NW = NC * NS
ROWS_PER_W = BATCH // NW
NCHUNK = ROWS_PER_W
TOK_PER_W = ROWS_PER_W * SEQ
LANES = 16
NBUF = 3
WIDE = 2 * D_MODEL
IDXPAD = SEQ + LANES


def _make_pe():
    pe = np.zeros((MAX_LEN, D_MODEL), dtype=np.float32)
    position = np.arange(0, MAX_LEN, dtype=np.float32)[:, None]
    div_term = np.exp(
        np.arange(0, D_MODEL, 2, dtype=np.float32) * (-math.log(10000.0) / D_MODEL)
    )
    pe[:, 0::2] = np.sin(position * div_term)
    pe[:, 1::2] = np.cos(position * div_term)
    return pe[:SEQ]


_PE_NP = _make_pe()

_MESH = plsc.VectorSubcoreMesh(
    core_axis_name="c", subcore_axis_name="s", num_cores=NC, num_subcores=NS
)


@functools.partial(
    pl.kernel,
    out_type=jax.ShapeDtypeStruct((BATCH, SEQ, WIDE), jnp.float32),
    mesh=_MESH,
    compiler_params=pltpu.CompilerParams(use_tc_tiling_on_sc=False),
    scratch_types=[
        pltpu.VMEM((IDXPAD,), jnp.int32),
        pltpu.VMEM((IDXPAD,), jnp.int32),
        pltpu.VMEM((IDXPAD,), jnp.int32),
        pltpu.VMEM((SEQ,), jnp.int32),
        pltpu.VMEM((SEQ,), jnp.int32),
        pltpu.VMEM((SEQ,), jnp.int32),
        pltpu.VMEM((SEQ, WIDE), jnp.float32),
        pltpu.VMEM((SEQ, WIDE), jnp.float32),
        pltpu.VMEM((SEQ, WIDE), jnp.float32),
        pltpu.VMEM((SEQ, D_MODEL), jnp.float32),
        pltpu.SemaphoreType.DMA,
        pltpu.SemaphoreType.DMA,
        pltpu.SemaphoreType.DMA,
        pltpu.SemaphoreType.DMA,
        pltpu.SemaphoreType.DMA,
        pltpu.SemaphoreType.DMA,
        pltpu.SemaphoreType.DMA,
        pltpu.SemaphoreType.DMA,
        pltpu.SemaphoreType.DMA,
    ],
)
def _embed_kernel(table2_hbm, xf_hbm, xfh_hbm, pe_hbm, out_hbm,
                  iv0, iv1, iv2, ih0, ih1, ih2, rows0, rows1, rows2, pe_v,
                  s0, s1, s2, g0, g1, g2, w0, w1, w2):
    wid = lax.axis_index("s") * NC + lax.axis_index("c")
    row0 = wid * ROWS_PER_W
    tok0 = row0 * SEQ

    idxv = [iv0, iv1, iv2]
    idxh = [ih0, ih1, ih2]
    rows = [rows0, rows1, rows2]
    isem = [s0, s1, s2]
    gsem = [g0, g1, g2]
    wsem = [w0, w1, w2]

    pltpu.sync_copy(pe_hbm, pe_v)

    def idxv_desc(c, k):
        return pltpu.make_async_copy(
            xf_hbm.at[pl.ds(tok0 + c * SEQ, SEQ)],
            idxv[k].at[pl.ds(0, SEQ)], isem[k],
        )

    def idxh_desc(c, k):
        return pltpu.make_async_copy(
            xfh_hbm.at[pl.ds(tok0 + c * SEQ, SEQ)], idxh[k], isem[k]
        )

    def fire_idx(c, k):
        idxv_desc(c, k).start()
        idxh_desc(c, k).start()

    def gather_desc(c, k):
        src = table2_hbm.at[idxh[k]]
        return pltpu.make_async_copy(src, rows[k], gsem[k])

    def fire_gather(c, k):
        idxv_desc(c, k).wait()
        idxh_desc(c, k).wait()
        gather_desc(c, k).start()

    def wb_desc(c, k):
        return pltpu.make_async_copy(rows[k], out_hbm.at[row0 + c], wsem[k])

    def process(c, k):
        gather_desc(c, k).wait()

        @plsc.parallel_loop(0, SEQ, unroll=2)
        def _ew(j):
            buf = rows[k]
            vvec = idxv[k][pl.ds(j, LANES)]
            odd = (vvec[0] & 1) == 1

            @pl.when(odd)
            def _():
                for q in range(D_MODEL // LANES):
                    sl = pl.ds(q * LANES, LANES)
                    hi = pl.ds(D_MODEL + q * LANES, LANES)
                    buf[j, sl] = buf[j, hi] * SCALE + pe_v[j, sl]

            @pl.when(jnp.logical_not(odd))
            def _():
                for q in range(D_MODEL // LANES):
                    sl = pl.ds(q * LANES, LANES)
                    buf[j, sl] = buf[j, sl] * SCALE + pe_v[j, sl]

        wb_desc(c, k).start()

    def drain_wb(c, k):
        wb_desc(c, k).wait()

    fire_idx(0, 0)
    fire_idx(1, 1)
    fire_idx(2, 2)
    fire_gather(0, 0)
    fire_gather(1, 1)

    process(0, 0)
    fire_idx(3, 0)
    fire_gather(2, 2)

    process(1, 1)
    fire_idx(4, 1)
    drain_wb(0, 0)
    fire_gather(3, 0)

    process(2, 2)
    fire_idx(5, 2)
    drain_wb(1, 1)
    fire_gather(4, 1)

    @pl.loop(1, (NCHUNK - 2) // NBUF)
    def _pipe(it):
        cbase = it * NBUF
        for k in range(NBUF):
            c = cbase + k
            process(c, k)

            @pl.when(c + 3 < NCHUNK)
            def _():
                fire_idx(c + 3, k)

            drain_wb(c - 1, (k + NBUF - 1) % NBUF)

            @pl.when(c + 2 < NCHUNK)
            def _():
                fire_gather(c + 2, (k + 2) % NBUF)

    process(NCHUNK - 2, (NCHUNK - 2) % NBUF)
    drain_wb(NCHUNK - 3, (NCHUNK - 3) % NBUF)
    process(NCHUNK - 1, (NCHUNK - 1) % NBUF)
    drain_wb(NCHUNK - 2, (NCHUNK - 2) % NBUF)
    drain_wb(NCHUNK - 1, (NCHUNK - 1) % NBUF)


def kernel(x, table):
    table2 = table.reshape(VOCAB // 2, WIDE)
    xf = x.reshape(BATCH * SEQ)
    xfh = lax.shift_right_logical(xf, 1)
    out = _embed_kernel(table2, xf, xfh, jnp.asarray(_PE_NP))
    return out[:, :, :D_MODEL]
